# baseline (speedup 1.0000x reference)
import functools

import jax
import jax.numpy as jnp
from jax import lax
from jax.experimental import pallas as pl
from jax.experimental.pallas import tpu as pltpu
from jax.experimental.pallas import tpu_sc as plsc

_I, _J = 8, 8192
_T_CH = 16


def _compute_chunk(ibuf, obuf, dbuf):
    def tile_body(t, carry):
        for g in range(8):
            dv = dbuf[pl.ds(t * 128 + g * 16, 16)]
            for s in range(8):
                obuf[t, s, pl.ds(g * 16, 16)] = (
                    ibuf[t, s, pl.ds(g * 16, 16)] * dv
                )
        return carry

    lax.fori_loop(0, _T_CH, tile_body, None)


def _sc_scale(a_hbm, d_hbm, o_hbm, ib0, ib1, ob0, ob1, db0, db1,
              ld0, ld1, st0, st1):
    info = plsc.get_sparse_core_info()
    nc = info.num_cores
    ns = info.num_subcores
    nw = nc * ns
    tiles_pw = (_I * _J) // nw
    n_chunks = tiles_pw // _T_CH
    n_pairs = n_chunks // 2
    wid = lax.axis_index("s") * nc + lax.axis_index("c")
    base = wid * tiles_pw
    i0 = base // _J
    j0 = base % _J

    half = _T_CH // 2

    def load(c, ib, db, sem):
        j = j0 + c * _T_CH
        pltpu.make_async_copy(
            a_hbm.at[i0, pl.ds(j, half)], ib.at[pl.ds(0, half)], sem).start()
        pltpu.make_async_copy(
            a_hbm.at[i0, pl.ds(j + half, half)],
            ib.at[pl.ds(half, half)], sem).start()
        pltpu.make_async_copy(
            d_hbm.at[pl.ds(j * 128, _T_CH * 128)], db, sem).start()

    def wait_load(ib, db, sem):
        pltpu.make_async_copy(
            a_hbm.at[i0, pl.ds(0, half)], ib.at[pl.ds(0, half)], sem).wait()
        pltpu.make_async_copy(
            a_hbm.at[i0, pl.ds(0, half)], ib.at[pl.ds(half, half)], sem).wait()
        pltpu.make_async_copy(
            d_hbm.at[pl.ds(0, _T_CH * 128)], db, sem).wait()

    def store(c, ob, sem):
        j = j0 + c * _T_CH
        pltpu.make_async_copy(
            ob.at[pl.ds(0, half)], o_hbm.at[i0, pl.ds(j, half)], sem).start()
        pltpu.make_async_copy(
            ob.at[pl.ds(half, half)],
            o_hbm.at[i0, pl.ds(j + half, half)], sem).start()

    def wait_store(ob, sem):
        pltpu.make_async_copy(
            ob.at[pl.ds(0, half)], o_hbm.at[i0, pl.ds(0, half)], sem).wait()
        pltpu.make_async_copy(
            ob.at[pl.ds(half, half)],
            o_hbm.at[i0, pl.ds(0, half)], sem).wait()

    load(0, ib0, db0, ld0)
    load(1, ib1, db1, ld1)

    def pair_body(kk, carry):
        c0 = kk * 2
        wait_load(ib0, db0, ld0)

        @pl.when(kk > 0)
        def _():
            wait_store(ob0, st0)

        _compute_chunk(ib0, ob0, db0)
        store(c0, ob0, st0)

        @pl.when(kk < n_pairs - 1)
        def _():
            load(c0 + 2, ib0, db0, ld0)

        wait_load(ib1, db1, ld1)

        @pl.when(kk > 0)
        def _():
            wait_store(ob1, st1)

        _compute_chunk(ib1, ob1, db1)
        store(c0 + 1, ob1, st1)

        @pl.when(kk < n_pairs - 1)
        def _():
            load(c0 + 3, ib1, db1, ld1)

        return carry

    lax.fori_loop(0, n_pairs, pair_body, None)
    wait_store(ob0, st0)
    wait_store(ob1, st1)


def kernel(x, diag):
    D, C = x.shape
    a = x.reshape(_J, 128, _I, 8).transpose(2, 0, 3, 1)
    mesh = plsc.VectorSubcoreMesh(core_axis_name="c", subcore_axis_name="s")
    run = functools.partial(
        pl.kernel,
        mesh=mesh,
        out_type=jax.ShapeDtypeStruct((_I, _J, 8, 128), jnp.float32),
        scratch_types=[
            pltpu.VMEM((_T_CH, 8, 128), jnp.float32),
            pltpu.VMEM((_T_CH, 8, 128), jnp.float32),
            pltpu.VMEM((_T_CH, 8, 128), jnp.float32),
            pltpu.VMEM((_T_CH, 8, 128), jnp.float32),
            pltpu.VMEM((_T_CH * 128,), jnp.float32),
            pltpu.VMEM((_T_CH * 128,), jnp.float32),
            pltpu.SemaphoreType.DMA,
            pltpu.SemaphoreType.DMA,
            pltpu.SemaphoreType.DMA,
            pltpu.SemaphoreType.DMA,
        ],
    )(_sc_scale)
    b = run(a, diag)
    return b.transpose(1, 3, 0, 2).reshape(D, C)


# speedup vs baseline: 1.0632x; 1.0632x over previous
import functools

import jax
import jax.numpy as jnp
from jax import lax
from jax.experimental import pallas as pl
from jax.experimental.pallas import tpu as pltpu
from jax.experimental.pallas import tpu_sc as plsc

_I, _J = 8, 8192
_T_CH = 16


def _compute_chunk(jc, ibuf, obuf, dbuf):
    def tile_body(t, carry):
        for g in range(8):
            dv = dbuf[pl.ds((jc * _T_CH + t) * 128 + g * 16, 16)]
            for s in range(8):
                obuf[t, s, pl.ds(g * 16, 16)] = (
                    ibuf[t, s, pl.ds(g * 16, 16)] * dv
                )
        return carry

    lax.fori_loop(0, _T_CH, tile_body, None)


def _sc_scale(a_hbm, d_hbm, o_hbm, ib0, ib1, ob0, ob1, dbuf,
              ld0, ld1, st0, st1):
    info = plsc.get_sparse_core_info()
    nc = info.num_cores
    ns = info.num_subcores
    nw = nc * ns
    j_pw = _J // nw
    n_jc = j_pw // _T_CH
    n_chunks = _I * n_jc
    n_pairs = n_chunks // 2
    wid = lax.axis_index("s") * nc + lax.axis_index("c")
    jbase = wid * j_pw

    def load(c, ib, sem):
        i = c // n_jc
        j = jbase + (c % n_jc) * _T_CH
        pltpu.make_async_copy(a_hbm.at[i, pl.ds(j, _T_CH)], ib, sem).start()

    def wait_load(ib, sem):
        pltpu.make_async_copy(a_hbm.at[0, pl.ds(0, _T_CH)], ib, sem).wait()

    def store(c, ob, sem):
        i = c // n_jc
        j = jbase + (c % n_jc) * _T_CH
        pltpu.make_async_copy(ob, o_hbm.at[i, pl.ds(j, _T_CH)], sem).start()

    def wait_store(ob, sem):
        pltpu.make_async_copy(ob, o_hbm.at[0, pl.ds(0, _T_CH)], sem).wait()

    pltpu.sync_copy(d_hbm.at[pl.ds(jbase * 128, j_pw * 128)], dbuf)

    load(0, ib0, ld0)
    load(1, ib1, ld1)

    def pair_body(kk, carry):
        c0 = kk * 2
        jc0 = c0 % n_jc
        wait_load(ib0, ld0)

        @pl.when(kk > 0)
        def _():
            wait_store(ob0, st0)

        _compute_chunk(jc0, ib0, ob0, dbuf)
        store(c0, ob0, st0)

        @pl.when(kk < n_pairs - 1)
        def _():
            load(c0 + 2, ib0, ld0)

        wait_load(ib1, ld1)

        @pl.when(kk > 0)
        def _():
            wait_store(ob1, st1)

        _compute_chunk(jc0 + 1, ib1, ob1, dbuf)
        store(c0 + 1, ob1, st1)

        @pl.when(kk < n_pairs - 1)
        def _():
            load(c0 + 3, ib1, ld1)

        return carry

    lax.fori_loop(0, n_pairs, pair_body, None)
    wait_store(ob0, st0)
    wait_store(ob1, st1)


def kernel(x, diag):
    D, C = x.shape
    a = x.reshape(_J, 128, _I, 8).transpose(2, 0, 3, 1)
    mesh = plsc.VectorSubcoreMesh(core_axis_name="c", subcore_axis_name="s")
    run = functools.partial(
        pl.kernel,
        mesh=mesh,
        out_type=jax.ShapeDtypeStruct((_I, _J, 8, 128), jnp.float32),
        scratch_types=[
            pltpu.VMEM((_T_CH, 8, 128), jnp.float32),
            pltpu.VMEM((_T_CH, 8, 128), jnp.float32),
            pltpu.VMEM((_T_CH, 8, 128), jnp.float32),
            pltpu.VMEM((_T_CH, 8, 128), jnp.float32),
            pltpu.VMEM((_J // 32 * 128,), jnp.float32),
            pltpu.SemaphoreType.DMA,
            pltpu.SemaphoreType.DMA,
            pltpu.SemaphoreType.DMA,
            pltpu.SemaphoreType.DMA,
        ],
    )(_sc_scale)
    b = run(a, diag)
    return b.transpose(1, 3, 0, 2).reshape(D, C)
